# Initial kernel scaffold; baseline (speedup 1.0000x reference)
import jax
import jax.numpy as jnp
from jax.experimental import pallas as pl


def kernel(x, edge_index, edge_attr, weight_matrix, bias):
    raise NotImplementedError("write your pallas kernel here")


# speedup vs baseline: 2.4008x; 2.4008x over previous
import functools

import jax
import jax.numpy as jnp
from jax import lax
from jax.experimental import pallas as pl
from jax.experimental.pallas import tpu as pltpu
from jax.experimental.pallas import tpu_sc as plsc

N_NODES = 10000
IN_CH = 16
OUT_CH = 16
NEA = 4
YW = 128

NC = 2
NS = 16
L = 16
NW = NC * NS
NG = NW // 2
HALF = N_NODES // 2
ACC_ROWS = HALF + 8

CHUNK = 128


def _ymat_body(x_ref, w_ref, o_ref):
    o_ref[...] = jnp.dot(x_ref[...], w_ref[...], preferred_element_type=jnp.float32)


def _combine_body(p_ref, b_ref, o_ref):
    rows = HALF * OUT_CH // 128
    parts = p_ref[...].reshape(NG, 2 * rows, 128)
    summed = jnp.sum(parts, axis=0)
    o_ref[...] = jnp.maximum(summed + b_ref[...], 0.0)


def _make_sc_kernel(epg):
    nchunk = epg // CHUNK
    mesh = plsc.VectorSubcoreMesh(core_axis_name="c", subcore_axis_name="s")

    @functools.partial(
        pl.kernel,
        mesh=mesh,
        out_type=jax.ShapeDtypeStruct((NW * HALF * OUT_CH,), jnp.float32),
        scratch_types=[
            pltpu.VMEM((CHUNK,), jnp.int32),
            pltpu.VMEM((CHUNK,), jnp.int32),
            pltpu.VMEM((CHUNK * NEA,), jnp.float32),
            pltpu.VMEM((CHUNK, YW), jnp.float32),
            pltpu.VMEM((ACC_ROWS * OUT_CH,), jnp.float32),
            pltpu.SemaphoreType.DMA,
        ],
    )
    def sc_kernel(y_hbm, src_hbm, dst_hbm, attr_hbm, out_hbm,
                  sidx, didx, attr_v, rows, acc, sem):
        c = lax.axis_index("c")
        s = lax.axis_index("s")
        wid = s * NC + c
        g = wid // 2
        lo = (wid % 2) * HALF

        def zbody(i, carry):
            acc[pl.ds(i * L, L)] = jnp.zeros((L,), jnp.float32)
            return carry

        lax.fori_loop(0, ACC_ROWS, zbody, 0)

        iota = lax.iota(jnp.int32, L)
        base = g * epg

        def chunk_body(i, carry):
            off = base + i * CHUNK
            pltpu.sync_copy(src_hbm.at[pl.ds(off, CHUNK)], sidx)
            pltpu.sync_copy(dst_hbm.at[pl.ds(off, CHUNK)], didx)
            pltpu.sync_copy(attr_hbm.at[pl.ds(off * NEA, CHUNK * NEA)], attr_v)
            pltpu.async_copy(y_hbm.at[sidx], rows, sem).wait()

            def ebody(q, ecarry):
                d16 = didx[pl.ds(q * L, L)]
                a0 = attr_v[pl.ds(q * (4 * L), L)]
                a1 = attr_v[pl.ds(q * (4 * L) + L, L)]
                a2 = attr_v[pl.ds(q * (4 * L) + 2 * L, L)]
                a3 = attr_v[pl.ds(q * (4 * L) + 3 * L, L)]
                avecs = (a0, a1, a2, a3)
                for u in range(L):
                    e = q * L + u
                    av = avecs[u // 4]
                    j0 = (u % 4) * 4
                    m = (av[j0] * rows[e, pl.ds(0, L)]
                         + av[j0 + 1] * rows[e, pl.ds(L, L)]
                         + av[j0 + 2] * rows[e, pl.ds(2 * L, L)]
                         + av[j0 + 3] * rows[e, pl.ds(3 * L, L)])
                    dst = d16[u]
                    rel = dst - lo
                    ok = (rel >= 0) & (rel < HALF)
                    row = jnp.where(ok, rel, HALF)
                    w = row * L
                    acc[pl.ds(w, L)] = acc[pl.ds(w, L)] + m
                return ecarry

            lax.fori_loop(0, CHUNK // L, ebody, 0)
            return carry

        lax.fori_loop(0, nchunk, chunk_body, 0)
        pltpu.sync_copy(acc.at[pl.ds(0, HALF * OUT_CH)],
                        out_hbm.at[pl.ds(wid * (HALF * OUT_CH), HALF * OUT_CH)])

    return sc_kernel


@jax.jit
def _run(x, src, dst, edge_attr, weight_matrix, bias):
    n_edges = src.shape[0]
    epg = -(-n_edges // (NG * CHUNK)) * CHUNK
    e_pad = epg * NG
    pad = e_pad - n_edges
    src_p = jnp.pad(src, (0, pad))
    dst_p = jnp.pad(dst, (0, pad))
    attr_p = jnp.pad(edge_attr, ((0, pad), (0, 0))).reshape(-1)

    wperm = weight_matrix.transpose(2, 0, 1).reshape(IN_CH, NEA * OUT_CH)
    wperm = jnp.pad(wperm, ((0, 0), (0, YW - NEA * OUT_CH)))
    y = pl.pallas_call(
        _ymat_body,
        out_shape=jax.ShapeDtypeStruct((N_NODES, YW), jnp.float32),
    )(x, wperm)

    rows = HALF * OUT_CH // 128
    parts = _make_sc_kernel(epg)(y, src_p, dst_p, attr_p).reshape(NW, rows, 128)
    bias_t = jnp.tile(bias, 128 // OUT_CH).reshape(1, 128)

    out = pl.pallas_call(
        _combine_body,
        out_shape=jax.ShapeDtypeStruct((2 * rows, 128), jnp.float32),
    )(parts, bias_t)
    return out.reshape(N_NODES, OUT_CH)


def kernel(x, edge_index, edge_attr, weight_matrix, bias):
    src = edge_index[0].astype(jnp.int32)
    dst = edge_index[1].astype(jnp.int32)
    return _run(x.astype(jnp.float32), src, dst,
                edge_attr.astype(jnp.float32),
                weight_matrix.astype(jnp.float32),
                bias.astype(jnp.float32))
